# baseline (speedup 1.0000x reference)
import functools

import jax
import jax.numpy as jnp
from jax import lax
from jax.experimental import pallas as pl
from jax.experimental.pallas import tpu as pltpu
from jax.experimental.pallas import tpu_sc as plsc

_N_USERS = 1000
_N_ITEMS = 1000
_HID = 512
_BATCH = 16384

_NC, _NS, _L = 2, 16, 16
_NW = _NC * _NS
_IDX_COLS = 128
_ROWS_PER_W = _BATCH // _IDX_COLS // _NW
_CNT = 1024
_ZPW = _CNT // _NS

_mesh = plsc.VectorSubcoreMesh(core_axis_name="c", subcore_axis_name="s")


@functools.partial(
    pl.kernel,
    mesh=_mesh,
    out_type=[
        jax.ShapeDtypeStruct((_NC, _CNT), jnp.float32),
        jax.ShapeDtypeStruct((_NC, _CNT), jnp.float32),
    ],
    scratch_types=[
        pltpu.VMEM((_ROWS_PER_W, _IDX_COLS), jnp.int32),
        pltpu.VMEM((_ROWS_PER_W, _IDX_COLS), jnp.int32),
        pltpu.VMEM((_IDX_COLS,), jnp.float32),
        pltpu.VMEM((_ZPW,), jnp.float32),
        pltpu.VMEM_SHARED((_CNT,), jnp.float32),
        pltpu.VMEM_SHARED((_CNT,), jnp.float32),
        pltpu.SemaphoreType.DMA,
        pltpu.SemaphoreType.DMA,
        pltpu.SemaphoreType.DMA,
    ],
)
def _hist_sc(users_ref, items_ref, ones_hbm, cu_out, ci_out,
             idxu_v, idxi_v, ones_v, zeros_v, shu, shi, semu, semi, sems):
    cid = lax.axis_index("c")
    sid = lax.axis_index("s")
    wid = sid * _NC + cid
    base = wid * _ROWS_PER_W * _IDX_COLS

    cps = [pltpu.async_copy(ones_hbm, ones_v, semu)]
    for j in range(_ROWS_PER_W):
        cps.append(pltpu.async_copy(
            users_ref.at[pl.ds(base + j * _IDX_COLS, _IDX_COLS)],
            idxu_v.at[j], semu))
        cps.append(pltpu.async_copy(
            items_ref.at[pl.ds(base + j * _IDX_COLS, _IDX_COLS)],
            idxi_v.at[j], semi))

    def _zbody(k, _):
        zeros_v[pl.ds(k * _L, _L)] = jnp.zeros((_L,), jnp.float32)
        return 0
    lax.fori_loop(0, _ZPW // _L, _zbody, 0)
    pltpu.sync_copy(zeros_v, shu.at[pl.ds(sid * _ZPW, _ZPW)])
    pltpu.sync_copy(zeros_v, shi.at[pl.ds(sid * _ZPW, _ZPW)])

    plsc.subcore_barrier()

    for c in cps:
        c.wait()
    scats = [pltpu.async_copy(ones_v, shu.at[idxu_v.at[j]], sems, add=True)
             for j in range(_ROWS_PER_W)]
    scats += [pltpu.async_copy(ones_v, shi.at[idxi_v.at[j]], sems, add=True)
              for j in range(_ROWS_PER_W)]
    for s in scats:
        s.wait()

    plsc.subcore_barrier()

    @pl.when(sid == 0)
    def _writeback():
        pltpu.sync_copy(shu, cu_out.at[cid])
        pltpu.sync_copy(shi, ci_out.at[cid])


def _rssq_body(ug_ref, ig_ref, ub_ref, ib_ref, ut_ref, uv_ref, out_ref):
    ug = ug_ref[...]
    ig = ig_ref[...]
    ut = ut_ref[...]
    uv = uv_ref[...]
    out_ref[0, :_N_USERS] = jnp.sum(ug * ug, axis=1)
    out_ref[1, :_N_ITEMS] = jnp.sum(ig * ig, axis=1)
    out_ref[2, :_N_USERS] = jnp.sum(ut * ut, axis=1)
    out_ref[3, :_N_USERS] = jnp.sum(uv * uv, axis=1)
    out_ref[4, :_N_USERS] = ub_ref[...][0, :] ** 2
    out_ref[5, :_N_ITEMS] = ib_ref[...][0, :] ** 2


_rssq_tc = pl.pallas_call(
    _rssq_body,
    out_shape=jax.ShapeDtypeStruct((8, _CNT), jnp.float32),
    compiler_params=pltpu.CompilerParams(
        vmem_limit_bytes=100 * 1024 * 1024,
    ),
)


def _combine_body(stats_ref, cu_ref, ci_ref, out_ref):
    cu = cu_ref[0, :_N_USERS] + cu_ref[1, :_N_USERS]
    ci = ci_ref[0, :_N_ITEMS] + ci_ref[1, :_N_ITEMS]
    present = cu > 0.0
    s_ug = jnp.sum(cu * stats_ref[0, :_N_USERS])
    s_ig = jnp.sum(ci * stats_ref[1, :_N_ITEMS])
    s_ut = jnp.sum(jnp.where(present, stats_ref[2, :_N_USERS], 0.0))
    s_uv = jnp.sum(jnp.where(present, stats_ref[3, :_N_USERS], 0.0))
    s_ub = jnp.sum(cu * stats_ref[4, :_N_USERS])
    s_ib = jnp.sum(ci * stats_ref[5, :_N_ITEMS])
    total = (jnp.sqrt(s_ug) + jnp.sqrt(s_ib) + jnp.sqrt(s_ub)
             + jnp.sqrt(s_ig) + jnp.sqrt(s_ut) + jnp.sqrt(s_uv))
    out_ref[...] = jnp.broadcast_to(total, (1, 1))


_combine_tc = pl.pallas_call(
    _combine_body,
    out_shape=jax.ShapeDtypeStruct((1, 1), jnp.float32),
    compiler_params=pltpu.CompilerParams(
        vmem_limit_bytes=100 * 1024 * 1024,
    ),
)


def kernel(users, items, user_gama, item_gama, user_beta, item_beta,
           theta_user_text, theta_user_visual):
    stats = _rssq_tc(user_gama, item_gama,
                     user_beta.reshape(1, _N_USERS),
                     item_beta.reshape(1, _N_ITEMS),
                     theta_user_text, theta_user_visual)
    ones = jnp.ones((_IDX_COLS,), jnp.float32)
    cu, ci = _hist_sc(users.astype(jnp.int32), items.astype(jnp.int32), ones)
    out = _combine_tc(stats, cu, ci)
    return out[0, 0]


# speedup vs baseline: 1.0381x; 1.0381x over previous
import functools

import jax
import jax.numpy as jnp
from jax import lax
from jax.experimental import pallas as pl
from jax.experimental.pallas import tpu as pltpu
from jax.experimental.pallas import tpu_sc as plsc

_N_USERS = 1000
_N_ITEMS = 1000
_HID = 512
_BATCH = 16384

_NC, _NS, _L = 2, 16, 16
_NW = _NC * _NS
_IDX_COLS = 128
_ROWS_PER_W = _BATCH // _IDX_COLS // _NW
_CNT = 1024
_ZPW = _CNT // _NS

_mesh = plsc.VectorSubcoreMesh(core_axis_name="c", subcore_axis_name="s")


@functools.partial(
    pl.kernel,
    mesh=_mesh,
    out_type=[
        jax.ShapeDtypeStruct((_NC, _CNT), jnp.float32),
        jax.ShapeDtypeStruct((_NC, _CNT), jnp.float32),
    ],
    scratch_types=[
        pltpu.VMEM((_ROWS_PER_W, _IDX_COLS), jnp.int32),
        pltpu.VMEM((_ROWS_PER_W, _IDX_COLS), jnp.int32),
        pltpu.VMEM((_IDX_COLS,), jnp.float32),
        pltpu.VMEM((_ZPW,), jnp.float32),
        pltpu.VMEM_SHARED((_CNT,), jnp.float32),
        pltpu.VMEM_SHARED((_CNT,), jnp.float32),
        pltpu.SemaphoreType.DMA,
        pltpu.SemaphoreType.DMA,
        pltpu.SemaphoreType.DMA,
    ],
)
def _hist_sc(users_ref, items_ref, cu_out, ci_out,
             idxu_v, idxi_v, ones_v, zeros_v, shu, shi, semu, semi, sems):
    cid = lax.axis_index("c")
    sid = lax.axis_index("s")
    wid = sid * _NC + cid
    base = wid * _ROWS_PER_W * _IDX_COLS

    cps = []
    for j in range(_ROWS_PER_W):
        cps.append(pltpu.async_copy(
            users_ref.at[pl.ds(base + j * _IDX_COLS, _IDX_COLS)],
            idxu_v.at[j], semu))
        cps.append(pltpu.async_copy(
            items_ref.at[pl.ds(base + j * _IDX_COLS, _IDX_COLS)],
            idxi_v.at[j], semi))

    def _obody(k, _):
        ones_v[pl.ds(k * _L, _L)] = jnp.full((_L,), 1.0, jnp.float32)
        return 0
    lax.fori_loop(0, _IDX_COLS // _L, _obody, 0)

    def _zbody(k, _):
        zeros_v[pl.ds(k * _L, _L)] = jnp.zeros((_L,), jnp.float32)
        return 0
    lax.fori_loop(0, _ZPW // _L, _zbody, 0)
    pltpu.sync_copy(zeros_v, shu.at[pl.ds(sid * _ZPW, _ZPW)])
    pltpu.sync_copy(zeros_v, shi.at[pl.ds(sid * _ZPW, _ZPW)])

    plsc.subcore_barrier()

    for c in cps:
        c.wait()
    scats = [pltpu.async_copy(ones_v, shu.at[idxu_v.at[j]], sems, add=True)
             for j in range(_ROWS_PER_W)]
    scats += [pltpu.async_copy(ones_v, shi.at[idxi_v.at[j]], sems, add=True)
              for j in range(_ROWS_PER_W)]
    for s in scats:
        s.wait()

    plsc.subcore_barrier()

    @pl.when(sid == 0)
    def _writeback():
        pltpu.sync_copy(shu, cu_out.at[cid])
        pltpu.sync_copy(shi, ci_out.at[cid])


def _rssq_body(ug_ref, ig_ref, ub_ref, ib_ref, ut_ref, uv_ref, out_ref):
    ug = ug_ref[...]
    ig = ig_ref[...]
    ut = ut_ref[...]
    uv = uv_ref[...]
    out_ref[0, :_N_USERS] = jnp.sum(ug * ug, axis=1)
    out_ref[1, :_N_ITEMS] = jnp.sum(ig * ig, axis=1)
    out_ref[2, :_N_USERS] = jnp.sum(ut * ut, axis=1)
    out_ref[3, :_N_USERS] = jnp.sum(uv * uv, axis=1)
    out_ref[4, :_N_USERS] = ub_ref[...][0, :] ** 2
    out_ref[5, :_N_ITEMS] = ib_ref[...][0, :] ** 2


_rssq_tc = pl.pallas_call(
    _rssq_body,
    out_shape=jax.ShapeDtypeStruct((8, _CNT), jnp.float32),
    compiler_params=pltpu.CompilerParams(
        vmem_limit_bytes=100 * 1024 * 1024,
    ),
)


def _combine_body(stats_ref, cu_ref, ci_ref, out_ref):
    cu = cu_ref[0, :_N_USERS] + cu_ref[1, :_N_USERS]
    ci = ci_ref[0, :_N_ITEMS] + ci_ref[1, :_N_ITEMS]
    present = cu > 0.0
    s_ug = jnp.sum(cu * stats_ref[0, :_N_USERS])
    s_ig = jnp.sum(ci * stats_ref[1, :_N_ITEMS])
    s_ut = jnp.sum(jnp.where(present, stats_ref[2, :_N_USERS], 0.0))
    s_uv = jnp.sum(jnp.where(present, stats_ref[3, :_N_USERS], 0.0))
    s_ub = jnp.sum(cu * stats_ref[4, :_N_USERS])
    s_ib = jnp.sum(ci * stats_ref[5, :_N_ITEMS])
    total = (jnp.sqrt(s_ug) + jnp.sqrt(s_ib) + jnp.sqrt(s_ub)
             + jnp.sqrt(s_ig) + jnp.sqrt(s_ut) + jnp.sqrt(s_uv))
    out_ref[...] = jnp.broadcast_to(total, (1, 1))


_combine_tc = pl.pallas_call(
    _combine_body,
    out_shape=jax.ShapeDtypeStruct((1, 1), jnp.float32),
    compiler_params=pltpu.CompilerParams(
        vmem_limit_bytes=100 * 1024 * 1024,
    ),
)


def kernel(users, items, user_gama, item_gama, user_beta, item_beta,
           theta_user_text, theta_user_visual):
    stats = _rssq_tc(user_gama, item_gama,
                     user_beta.reshape(1, _N_USERS),
                     item_beta.reshape(1, _N_ITEMS),
                     theta_user_text, theta_user_visual)
    cu, ci = _hist_sc(users.astype(jnp.int32), items.astype(jnp.int32))
    out = _combine_tc(stats, cu, ci)
    return out[0, 0]


# speedup vs baseline: 1.0802x; 1.0405x over previous
import functools

import jax
import jax.numpy as jnp
from jax import lax
from jax.experimental import pallas as pl
from jax.experimental.pallas import tpu as pltpu
from jax.experimental.pallas import tpu_sc as plsc

_N_USERS = 1000
_N_ITEMS = 1000
_HID = 512
_BATCH = 16384

_NC, _NS, _L = 1, 16, 16
_NW = _NC * _NS
_IDX_COLS = 128
_ROWS_PER_W = _BATCH // _IDX_COLS // _NW
_CNT = 1024
_ZPW = _CNT // _NS

_mesh = plsc.VectorSubcoreMesh(core_axis_name="c", subcore_axis_name="s",
                               num_cores=_NC)


@functools.partial(
    pl.kernel,
    mesh=_mesh,
    out_type=[
        jax.ShapeDtypeStruct((_NC, _CNT), jnp.float32),
        jax.ShapeDtypeStruct((_NC, _CNT), jnp.float32),
    ],
    scratch_types=[
        pltpu.VMEM((_ROWS_PER_W, _IDX_COLS), jnp.int32),
        pltpu.VMEM((_ROWS_PER_W, _IDX_COLS), jnp.int32),
        pltpu.VMEM((_IDX_COLS,), jnp.float32),
        pltpu.VMEM((_ZPW,), jnp.float32),
        pltpu.VMEM_SHARED((_CNT,), jnp.float32),
        pltpu.VMEM_SHARED((_CNT,), jnp.float32),
        pltpu.SemaphoreType.DMA,
        pltpu.SemaphoreType.DMA,
        pltpu.SemaphoreType.DMA,
    ],
)
def _hist_sc(users_ref, items_ref, cu_out, ci_out,
             idxu_v, idxi_v, ones_v, zeros_v, shu, shi, semu, semi, sems):
    cid = lax.axis_index("c")
    sid = lax.axis_index("s")
    wid = sid * _NC + cid
    base = wid * _ROWS_PER_W * _IDX_COLS

    cps = []
    for j in range(_ROWS_PER_W):
        cps.append(pltpu.async_copy(
            users_ref.at[pl.ds(base + j * _IDX_COLS, _IDX_COLS)],
            idxu_v.at[j], semu))
        cps.append(pltpu.async_copy(
            items_ref.at[pl.ds(base + j * _IDX_COLS, _IDX_COLS)],
            idxi_v.at[j], semi))

    def _obody(k, _):
        ones_v[pl.ds(k * _L, _L)] = jnp.full((_L,), 1.0, jnp.float32)
        return 0
    lax.fori_loop(0, _IDX_COLS // _L, _obody, 0)

    def _zbody(k, _):
        zeros_v[pl.ds(k * _L, _L)] = jnp.zeros((_L,), jnp.float32)
        return 0
    lax.fori_loop(0, _ZPW // _L, _zbody, 0)
    pltpu.sync_copy(zeros_v, shu.at[pl.ds(sid * _ZPW, _ZPW)])
    pltpu.sync_copy(zeros_v, shi.at[pl.ds(sid * _ZPW, _ZPW)])

    plsc.subcore_barrier()

    for c in cps:
        c.wait()
    scats = [pltpu.async_copy(ones_v, shu.at[idxu_v.at[j]], sems, add=True)
             for j in range(_ROWS_PER_W)]
    scats += [pltpu.async_copy(ones_v, shi.at[idxi_v.at[j]], sems, add=True)
              for j in range(_ROWS_PER_W)]
    for s in scats:
        s.wait()

    plsc.subcore_barrier()

    @pl.when(sid == 0)
    def _writeback():
        pltpu.sync_copy(shu, cu_out.at[cid])
        pltpu.sync_copy(shi, ci_out.at[cid])


def _rssq_body(ug_ref, ig_ref, ub_ref, ib_ref, ut_ref, uv_ref, out_ref):
    ug = ug_ref[...]
    ig = ig_ref[...]
    ut = ut_ref[...]
    uv = uv_ref[...]
    out_ref[0, :_N_USERS] = jnp.sum(ug * ug, axis=1)
    out_ref[1, :_N_ITEMS] = jnp.sum(ig * ig, axis=1)
    out_ref[2, :_N_USERS] = jnp.sum(ut * ut, axis=1)
    out_ref[3, :_N_USERS] = jnp.sum(uv * uv, axis=1)
    out_ref[4, :_N_USERS] = ub_ref[...][0, :] ** 2
    out_ref[5, :_N_ITEMS] = ib_ref[...][0, :] ** 2


_rssq_tc = pl.pallas_call(
    _rssq_body,
    out_shape=jax.ShapeDtypeStruct((8, _CNT), jnp.float32),
    compiler_params=pltpu.CompilerParams(
        vmem_limit_bytes=100 * 1024 * 1024,
    ),
)


def _combine_body(stats_ref, cu_ref, ci_ref, out_ref):
    cu = cu_ref[0, :_N_USERS]
    ci = ci_ref[0, :_N_ITEMS]
    for r in range(1, _NC):
        cu = cu + cu_ref[r, :_N_USERS]
        ci = ci + ci_ref[r, :_N_ITEMS]
    present = cu > 0.0
    s_ug = jnp.sum(cu * stats_ref[0, :_N_USERS])
    s_ig = jnp.sum(ci * stats_ref[1, :_N_ITEMS])
    s_ut = jnp.sum(jnp.where(present, stats_ref[2, :_N_USERS], 0.0))
    s_uv = jnp.sum(jnp.where(present, stats_ref[3, :_N_USERS], 0.0))
    s_ub = jnp.sum(cu * stats_ref[4, :_N_USERS])
    s_ib = jnp.sum(ci * stats_ref[5, :_N_ITEMS])
    total = (jnp.sqrt(s_ug) + jnp.sqrt(s_ib) + jnp.sqrt(s_ub)
             + jnp.sqrt(s_ig) + jnp.sqrt(s_ut) + jnp.sqrt(s_uv))
    out_ref[...] = jnp.broadcast_to(total, (1, 1))


_combine_tc = pl.pallas_call(
    _combine_body,
    out_shape=jax.ShapeDtypeStruct((1, 1), jnp.float32),
    compiler_params=pltpu.CompilerParams(
        vmem_limit_bytes=100 * 1024 * 1024,
    ),
)


def kernel(users, items, user_gama, item_gama, user_beta, item_beta,
           theta_user_text, theta_user_visual):
    stats = _rssq_tc(user_gama, item_gama,
                     user_beta.reshape(1, _N_USERS),
                     item_beta.reshape(1, _N_ITEMS),
                     theta_user_text, theta_user_visual)
    cu, ci = _hist_sc(users.astype(jnp.int32), items.astype(jnp.int32))
    out = _combine_tc(stats, cu, ci)
    return out[0, 0]


# speedup vs baseline: 1.0927x; 1.0115x over previous
import functools

import jax
import jax.numpy as jnp
from jax import lax
from jax.experimental import pallas as pl
from jax.experimental.pallas import tpu as pltpu
from jax.experimental.pallas import tpu_sc as plsc

_N_USERS = 1000
_N_ITEMS = 1000
_HID = 512
_BATCH = 16384

_NC, _NS, _L = 1, 16, 16
_NW = _NC * _NS
_IDX_COLS = 128
_ROWS_PER_W = _BATCH // _IDX_COLS // _NW
_CNT = 1024
_ZPW = _CNT // _NS

_mesh = plsc.VectorSubcoreMesh(core_axis_name="c", subcore_axis_name="s",
                               num_cores=_NC)


@functools.partial(
    pl.kernel,
    mesh=_mesh,
    out_type=[
        jax.ShapeDtypeStruct((_NC, _CNT), jnp.float32),
        jax.ShapeDtypeStruct((_NC, _CNT), jnp.float32),
    ],
    scratch_types=[
        pltpu.VMEM((_ROWS_PER_W, _IDX_COLS), jnp.int32),
        pltpu.VMEM((_ROWS_PER_W, _IDX_COLS), jnp.int32),
        pltpu.VMEM((_IDX_COLS,), jnp.float32),
        pltpu.VMEM((_ZPW,), jnp.float32),
        pltpu.VMEM_SHARED((_CNT,), jnp.float32),
        pltpu.VMEM_SHARED((_CNT,), jnp.float32),
        pltpu.SemaphoreType.DMA,
        pltpu.SemaphoreType.DMA,
        pltpu.SemaphoreType.DMA,
    ],
)
def _hist_sc(users_ref, items_ref, cu_out, ci_out,
             idxu_v, idxi_v, ones_v, zeros_v, shu, shi, semu, semi, sems):
    cid = lax.axis_index("c")
    sid = lax.axis_index("s")
    wid = sid * _NC + cid
    base = wid * _ROWS_PER_W

    cp_u = pltpu.async_copy(users_ref.at[pl.ds(base, _ROWS_PER_W)], idxu_v,
                            semu)
    cp_i = pltpu.async_copy(items_ref.at[pl.ds(base, _ROWS_PER_W)], idxi_v,
                            semi)

    def _obody(k, _):
        ones_v[pl.ds(k * _L, _L)] = jnp.full((_L,), 1.0, jnp.float32)
        return 0
    lax.fori_loop(0, _IDX_COLS // _L, _obody, 0)

    def _zbody(k, _):
        zeros_v[pl.ds(k * _L, _L)] = jnp.zeros((_L,), jnp.float32)
        return 0
    lax.fori_loop(0, _ZPW // _L, _zbody, 0)
    pltpu.sync_copy(zeros_v, shu.at[pl.ds(sid * _ZPW, _ZPW)])
    pltpu.sync_copy(zeros_v, shi.at[pl.ds(sid * _ZPW, _ZPW)])

    plsc.subcore_barrier()

    cp_u.wait()
    scats = [pltpu.async_copy(ones_v, shu.at[idxu_v.at[j]], sems, add=True)
             for j in range(_ROWS_PER_W)]
    cp_i.wait()
    scats += [pltpu.async_copy(ones_v, shi.at[idxi_v.at[j]], sems, add=True)
              for j in range(_ROWS_PER_W)]
    for s in scats:
        s.wait()

    plsc.subcore_barrier()

    @pl.when(sid == 0)
    def _writeback():
        pltpu.sync_copy(shu, cu_out.at[cid])
        pltpu.sync_copy(shi, ci_out.at[cid])


def _rssq_body(ug_ref, ig_ref, ub_ref, ib_ref, ut_ref, uv_ref, out_ref):
    ug = ug_ref[...]
    ig = ig_ref[...]
    ut = ut_ref[...]
    uv = uv_ref[...]
    out_ref[0, :_N_USERS] = jnp.sum(ug * ug, axis=1)
    out_ref[1, :_N_ITEMS] = jnp.sum(ig * ig, axis=1)
    out_ref[2, :_N_USERS] = jnp.sum(ut * ut, axis=1)
    out_ref[3, :_N_USERS] = jnp.sum(uv * uv, axis=1)
    out_ref[4, :_N_USERS] = ub_ref[...][0, :] ** 2
    out_ref[5, :_N_ITEMS] = ib_ref[...][0, :] ** 2


_rssq_tc = pl.pallas_call(
    _rssq_body,
    out_shape=jax.ShapeDtypeStruct((8, _CNT), jnp.float32),
    compiler_params=pltpu.CompilerParams(
        vmem_limit_bytes=100 * 1024 * 1024,
    ),
)


def _combine_body(stats_ref, cu_ref, ci_ref, out_ref):
    cu = cu_ref[0, :_N_USERS]
    ci = ci_ref[0, :_N_ITEMS]
    for r in range(1, _NC):
        cu = cu + cu_ref[r, :_N_USERS]
        ci = ci + ci_ref[r, :_N_ITEMS]
    present = cu > 0.0
    s_ug = jnp.sum(cu * stats_ref[0, :_N_USERS])
    s_ig = jnp.sum(ci * stats_ref[1, :_N_ITEMS])
    s_ut = jnp.sum(jnp.where(present, stats_ref[2, :_N_USERS], 0.0))
    s_uv = jnp.sum(jnp.where(present, stats_ref[3, :_N_USERS], 0.0))
    s_ub = jnp.sum(cu * stats_ref[4, :_N_USERS])
    s_ib = jnp.sum(ci * stats_ref[5, :_N_ITEMS])
    total = (jnp.sqrt(s_ug) + jnp.sqrt(s_ib) + jnp.sqrt(s_ub)
             + jnp.sqrt(s_ig) + jnp.sqrt(s_ut) + jnp.sqrt(s_uv))
    out_ref[...] = jnp.broadcast_to(total, (1, 1))


_combine_tc = pl.pallas_call(
    _combine_body,
    out_shape=jax.ShapeDtypeStruct((1, 1), jnp.float32),
    compiler_params=pltpu.CompilerParams(
        vmem_limit_bytes=100 * 1024 * 1024,
    ),
)


def kernel(users, items, user_gama, item_gama, user_beta, item_beta,
           theta_user_text, theta_user_visual):
    stats = _rssq_tc(user_gama, item_gama,
                     user_beta.reshape(1, _N_USERS),
                     item_beta.reshape(1, _N_ITEMS),
                     theta_user_text, theta_user_visual)
    users2d = users.astype(jnp.int32).reshape(-1, _IDX_COLS)
    items2d = items.astype(jnp.int32).reshape(-1, _IDX_COLS)
    cu, ci = _hist_sc(users2d, items2d)
    out = _combine_tc(stats, cu, ci)
    return out[0, 0]
